# baseline (speedup 1.0000x reference)
import functools

import jax
import jax.numpy as jnp
from jax import lax
from jax.experimental import pallas as pl
from jax.experimental.pallas import tpu as pltpu
from jax.experimental.pallas import tpu_sc as plsc

N = 10000
E = 320000
H = 128
G = 64

NC = 2
NS = 16
NW = NC * NS
EPW = 10240
E_PAD = NW * EPW
CHUNK = 64
NCHUNKS = EPW // CHUNK
NSLAB = 4
HCHUNKS = NCHUNKS // NSLAB
NBUF = 4
NPAD = 10112
ZROWS = NPAD // NS
OROWS = 632

_sc_mesh = plsc.VectorSubcoreMesh(core_axis_name="c", subcore_axis_name="s")


@functools.partial(
    pl.kernel,
    out_type=jax.ShapeDtypeStruct((NC, N, H), jnp.float32),
    mesh=_sc_mesh,
    scratch_types=[
        pltpu.VMEM((HCHUNKS, CHUNK), jnp.int32),
        pltpu.VMEM((HCHUNKS, CHUNK), jnp.int32),
        [pltpu.VMEM((CHUNK, H), jnp.float32)] * NBUF,
        [pltpu.SemaphoreType.DMA] * NBUF,
        [pltpu.SemaphoreType.DMA] * NBUF,
        pltpu.VMEM_SHARED((NPAD, H), jnp.float32),
    ],
)
def _sc_aggregate(z_hbm, src_hbm, dst_hbm, zeros_hbm, out_hbm,
                  sidx, didx, rows, gsems, ssems, agg):
    cid = lax.axis_index("c")
    sid = lax.axis_index("s")
    wid = cid * NS + sid

    pltpu.sync_copy(zeros_hbm.at[pl.ds(sid * ZROWS, ZROWS)],
                    agg.at[pl.ds(sid * ZROWS, ZROWS)])
    plsc.subcore_barrier()

    def gather(c, b):
        pltpu.async_copy(z_hbm.at[sidx.at[c]], rows[b], gsems[b])

    def gather_wait(c, b):
        pltpu.make_async_copy(z_hbm.at[sidx.at[c]], rows[b], gsems[b]).wait()

    def scatter(c, b):
        pltpu.async_copy(rows[b], agg.at[didx.at[c]], ssems[b], add=True)

    def scatter_wait(c, b):
        pltpu.make_async_copy(rows[b], agg.at[didx.at[c]], ssems[b]).wait()

    for slab in range(NSLAB):
        pltpu.sync_copy(src_hbm.at[wid, pl.ds(slab * HCHUNKS, HCHUNKS)], sidx)
        pltpu.sync_copy(dst_hbm.at[wid, pl.ds(slab * HCHUNKS, HCHUNKS)], didx)

        def step(c, bufc, first=False, last=False):
            if not first:
                scatter_wait(c - 2, (bufc - 2) % NBUF)
            if not last:
                gather(c + 2, (bufc + 2) % NBUF)
            gather_wait(c, bufc)
            scatter(c, bufc)

        gather(0, 0)
        gather(1, 1)
        step(0, 0, first=True)
        step(1, 1, first=True)

        def body(i, carry):
            for b in range(NBUF):
                step(4 * i + 2 + b, (2 + b) % NBUF)
            return carry

        lax.fori_loop(0, (HCHUNKS - 4) // NBUF, body, 0)

        step(HCHUNKS - 2, (HCHUNKS - 2) % NBUF, last=True)
        step(HCHUNKS - 1, (HCHUNKS - 1) % NBUF, last=True)
        scatter_wait(HCHUNKS - 2, (HCHUNKS - 2) % NBUF)
        scatter_wait(HCHUNKS - 1, (HCHUNKS - 1) % NBUF)

    plsc.subcore_barrier()

    ostart = jnp.minimum(sid * OROWS, N - OROWS)
    pltpu.sync_copy(agg.at[pl.ds(ostart, OROWS)],
                    out_hbm.at[cid, pl.ds(ostart, OROWS)])


R = 2000
GRID = N // R


def _layer_body(z_ref, p_ref, w1_ref, b1_ref, w2_ref, b2_ref,
                gamma_ref, beta_ref, batch_ref,
                zout_ref, g_ref, u_scr, stat_scr, gacc_ref):
    ph = pl.program_id(0)
    i = pl.program_id(1)

    @pl.when(ph == 0)
    def _p0():
        h = z_ref[...] + p_ref[0] + p_ref[1]
        h = jnp.maximum(
            jnp.dot(h, w1_ref[...], preferred_element_type=jnp.float32)
            + b1_ref[...], 0.0)
        h = jnp.dot(h, w2_ref[...],
                    preferred_element_type=jnp.float32) + b2_ref[...]
        u = jnp.maximum(h, 0.0)
        u_scr[pl.ds(i * R, R), :] = u

        @pl.when(i == 0)
        def _init():
            stat_scr[...] = jnp.zeros_like(stat_scr)

        stat_scr[0:1, :] += jnp.sum(u, axis=0, keepdims=True)
        stat_scr[1:2, :] += jnp.sum(u * u, axis=0, keepdims=True)

    @pl.when(ph == 1)
    def _p1():
        sums = stat_scr[...]
        mean = sums[0:1, :] * (1.0 / N)
        var = sums[1:2, :] * (1.0 / N) - mean * mean
        scale = gamma_ref[...] / jnp.sqrt(var + 1e-5)
        shift = beta_ref[...] - mean * scale
        zb = u_scr[pl.ds(i * R, R), :] * scale + shift
        zout_ref[...] = zb

        b = batch_ref[...]
        onehot = (b == lax.broadcasted_iota(jnp.int32,
                                            (R, G), 1)).astype(jnp.float32)

        @pl.when(i == 0)
        def _init():
            gacc_ref[...] = jnp.zeros_like(gacc_ref)

        gacc_ref[...] += lax.dot_general(onehot, zb, (((0,), (0,)), ((), ())),
                                         preferred_element_type=jnp.float32)

        @pl.when(i == GRID - 1)
        def _fin():
            g_ref[...] = gacc_ref[...]


_tc_layer = pl.pallas_call(
    _layer_body,
    grid=(2, GRID),
    in_specs=[
        pl.BlockSpec((R, H), lambda p, i: (i * (1 - p), 0)),
        pl.BlockSpec((NC, R, H), lambda p, i: (0, i * (1 - p), 0)),
        pl.BlockSpec((H, H), lambda p, i: (0, 0)),
        pl.BlockSpec((1, H), lambda p, i: (0, 0)),
        pl.BlockSpec((H, H), lambda p, i: (0, 0)),
        pl.BlockSpec((1, H), lambda p, i: (0, 0)),
        pl.BlockSpec((1, H), lambda p, i: (0, 0)),
        pl.BlockSpec((1, H), lambda p, i: (0, 0)),
        pl.BlockSpec((R, 1), lambda p, i: (i * p, 0)),
    ],
    out_specs=[
        pl.BlockSpec((R, H), lambda p, i: (i * p, 0)),
        pl.BlockSpec((G, H), lambda p, i: (0, 0)),
    ],
    out_shape=[
        jax.ShapeDtypeStruct((N, H), jnp.float32),
        jax.ShapeDtypeStruct((G, H), jnp.float32),
    ],
    scratch_shapes=[
        pltpu.VMEM((N, H), jnp.float32),
        pltpu.VMEM((8, H), jnp.float32),
        pltpu.VMEM((G, H), jnp.float32),
    ],
)


def kernel(x, edge_index, batch, params):
    src = edge_index[0]
    dst = edge_index[1]
    pad = E_PAD - E
    pad_ids = jnp.arange(pad, dtype=jnp.int32)
    src_p = jnp.concatenate([src, pad_ids % N])
    dst_p = jnp.concatenate([dst, N + pad_ids % (NPAD - N)])
    src_all = src_p.reshape(NW, NCHUNKS, CHUNK)
    dst_all = dst_p.reshape(NW, NCHUNKS, CHUNK)
    zeros_init = jnp.zeros((NPAD, H), jnp.float32)
    batch2 = batch.reshape(N, 1)

    z = x
    zs = []
    gs = []
    for (w1, b1, w2, b2, gamma, beta) in params:
        p = _sc_aggregate(z, src_all, dst_all, zeros_init)
        z, g = _tc_layer(z, p, w1, b1.reshape(1, H), w2, b2.reshape(1, H),
                         gamma.reshape(1, H), beta.reshape(1, H), batch2)
        zs.append(z)
        gs.append(g)
    return jnp.concatenate(zs, axis=1), jnp.concatenate(gs, axis=1)


# speedup vs baseline: 1.0031x; 1.0031x over previous
import functools

import jax
import jax.numpy as jnp
from jax import lax
from jax.experimental import pallas as pl
from jax.experimental.pallas import tpu as pltpu
from jax.experimental.pallas import tpu_sc as plsc

N = 10000
E = 320000
H = 128
G = 64

NC = 2
NS = 16
NW = NC * NS
EPW = 10240
E_PAD = NW * EPW
CHUNK = 128
NCHUNKS = EPW // CHUNK
NSLAB = 2
HCHUNKS = NCHUNKS // NSLAB
NBUF = 2
NPAD = 10112
ZROWS = NPAD // NS
OROWS = 632

_sc_mesh = plsc.VectorSubcoreMesh(core_axis_name="c", subcore_axis_name="s")


def _make_sc_aggregate(ncols, col0):
    @functools.partial(
        pl.kernel,
        out_type=jax.ShapeDtypeStruct((NC, N, H), jnp.float32),
        mesh=_sc_mesh,
        scratch_types=[
            pltpu.VMEM((HCHUNKS, CHUNK), jnp.int32),
            pltpu.VMEM((HCHUNKS, CHUNK), jnp.int32),
            [pltpu.VMEM((CHUNK, H), jnp.float32)] * NBUF,
            [pltpu.SemaphoreType.DMA] * NBUF,
            [pltpu.SemaphoreType.DMA] * NBUF,
            pltpu.VMEM_SHARED((NPAD, H), jnp.float32),
        ],
        name=f"sc_aggregate_c{col0}",
    )
    def _sc_aggregate(z_hbm, src_hbm, dst_hbm, zeros_hbm, out_hbm,
                      sidx, didx, rows, gsems, ssems, agg):
        cid = lax.axis_index("c")
        sid = lax.axis_index("s")
        wid = cid * NS + sid

        pltpu.sync_copy(zeros_hbm.at[pl.ds(sid * ZROWS, ZROWS)],
                        agg.at[pl.ds(sid * ZROWS, ZROWS)])
        plsc.subcore_barrier()

        def zsrc(c):
            if ncols == H:
                return z_hbm.at[sidx.at[c]]
            return z_hbm.at[sidx.at[c], pl.ds(col0, H)]

        def gather(c, b):
            pltpu.async_copy(zsrc(c), rows[b], gsems[b])

        def gather_wait(c, b):
            pltpu.make_async_copy(zsrc(c), rows[b], gsems[b]).wait()

        def scatter(c, b):
            pltpu.async_copy(rows[b], agg.at[didx.at[c]], ssems[b], add=True)

        def scatter_wait(c, b):
            pltpu.make_async_copy(rows[b], agg.at[didx.at[c]],
                                  ssems[b]).wait()

        for slab in range(NSLAB):
            pltpu.sync_copy(src_hbm.at[wid, pl.ds(slab * HCHUNKS, HCHUNKS)],
                            sidx)
            pltpu.sync_copy(dst_hbm.at[wid, pl.ds(slab * HCHUNKS, HCHUNKS)],
                            didx)

            def step(c, b, first=False, last=False):
                bn = 1 - b
                if not first:
                    scatter_wait(c - 1, bn)
                if not last:
                    gather(c + 1, bn)
                gather_wait(c, b)
                scatter(c, b)

            gather(0, 0)
            step(0, 0, first=True)
            step(1, 1)

            def body(i, carry):
                step(2 * i, 0)
                step(2 * i + 1, 1)
                return carry

            lax.fori_loop(1, HCHUNKS // 2 - 1, body, 0)

            step(HCHUNKS - 2, 0)
            step(HCHUNKS - 1, 1, last=True)
            scatter_wait(HCHUNKS - 1, 1)

        plsc.subcore_barrier()

        ostart = jnp.minimum(sid * OROWS, N - OROWS)
        pltpu.sync_copy(agg.at[pl.ds(ostart, OROWS)],
                        out_hbm.at[cid, pl.ds(ostart, OROWS)])

    return _sc_aggregate


_sc_agg_x = _make_sc_aggregate(H, 0)
_sc_agg_cat = [_make_sc_aggregate(3 * H, 0), _make_sc_aggregate(3 * H, H)]


R = 2000
GRID = N // R


def _layer_compute(z_ref, p_ref, w1_ref, b1_ref, w2_ref, b2_ref,
                   gamma_ref, beta_ref, batch_ref,
                   zout_ref, g_ref, u_scr, stat_scr, gacc_ref):
    ph = pl.program_id(0)
    i = pl.program_id(1)

    @pl.when(ph == 0)
    def _p0():
        h = z_ref[...] + p_ref[0] + p_ref[1]
        h = jnp.maximum(
            jnp.dot(h, w1_ref[...], preferred_element_type=jnp.float32)
            + b1_ref[...], 0.0)
        h = jnp.dot(h, w2_ref[...],
                    preferred_element_type=jnp.float32) + b2_ref[...]
        u = jnp.maximum(h, 0.0)
        u_scr[pl.ds(i * R, R), :] = u

        @pl.when(i == 0)
        def _init():
            stat_scr[...] = jnp.zeros_like(stat_scr)

        stat_scr[0:1, :] += jnp.sum(u, axis=0, keepdims=True)
        stat_scr[1:2, :] += jnp.sum(u * u, axis=0, keepdims=True)

    @pl.when(ph == 1)
    def _p1():
        sums = stat_scr[...]
        mean = sums[0:1, :] * (1.0 / N)
        var = sums[1:2, :] * (1.0 / N) - mean * mean
        scale = gamma_ref[...] / jnp.sqrt(var + 1e-5)
        shift = beta_ref[...] - mean * scale
        zb = u_scr[pl.ds(i * R, R), :] * scale + shift
        zout_ref[...] = zb

        b = batch_ref[...]
        onehot = (b == lax.broadcasted_iota(jnp.int32,
                                            (R, G), 1)).astype(jnp.float32)

        @pl.when(i == 0)
        def _init():
            gacc_ref[...] = jnp.zeros_like(gacc_ref)

        gacc_ref[...] += lax.dot_general(onehot, zb, (((0,), (0,)), ((), ())),
                                         preferred_element_type=jnp.float32)

        @pl.when(i == GRID - 1)
        def _fin():
            g_ref[...] = gacc_ref[...]


def _make_tc_layer(l):
    read_cat = l > 0

    def body(*refs):
        if read_cat:
            (zcat_ref, p_ref, w1_ref, b1_ref, w2_ref, b2_ref,
             gamma_ref, beta_ref, batch_ref, zout_ref, g_ref,
             u_scr, stat_scr, gacc_ref) = refs
            z_ref = zcat_ref
        else:
            (zcat_ref, z_ref, p_ref, w1_ref, b1_ref, w2_ref, b2_ref,
             gamma_ref, beta_ref, batch_ref, zout_ref, g_ref,
             u_scr, stat_scr, gacc_ref) = refs
        _layer_compute(z_ref, p_ref, w1_ref, b1_ref, w2_ref, b2_ref,
                       gamma_ref, beta_ref, batch_ref,
                       zout_ref, g_ref, u_scr, stat_scr, gacc_ref)

    if read_cat:
        zcat_spec = pl.BlockSpec((R, H), lambda p, i: (i * (1 - p), l - 1))
        zin_specs = []
    else:
        zcat_spec = pl.BlockSpec((8, H), lambda p, i: (0, 0))
        zin_specs = [pl.BlockSpec((R, H), lambda p, i: (i * (1 - p), 0))]

    return pl.pallas_call(
        body,
        grid=(2, GRID),
        in_specs=[zcat_spec] + zin_specs + [
            pl.BlockSpec((NC, R, H), lambda p, i: (0, i * (1 - p), 0)),
            pl.BlockSpec((H, H), lambda p, i: (0, 0)),
            pl.BlockSpec((1, H), lambda p, i: (0, 0)),
            pl.BlockSpec((H, H), lambda p, i: (0, 0)),
            pl.BlockSpec((1, H), lambda p, i: (0, 0)),
            pl.BlockSpec((1, H), lambda p, i: (0, 0)),
            pl.BlockSpec((1, H), lambda p, i: (0, 0)),
            pl.BlockSpec((R, 1), lambda p, i: (i * p, 0)),
        ],
        out_specs=[
            pl.BlockSpec((R, H), lambda p, i: (i * p, l)),
            pl.BlockSpec((G, H), lambda p, i: (0, 0)),
        ],
        out_shape=[
            jax.ShapeDtypeStruct((N, 3 * H), jnp.float32),
            jax.ShapeDtypeStruct((G, H), jnp.float32),
        ],
        scratch_shapes=[
            pltpu.VMEM((N, H), jnp.float32),
            pltpu.VMEM((8, H), jnp.float32),
            pltpu.VMEM((G, H), jnp.float32),
        ],
        input_output_aliases={0: 0},
    )


_tc_layers = [_make_tc_layer(l) for l in range(3)]


def kernel(x, edge_index, batch, params):
    src = edge_index[0]
    dst = edge_index[1]
    pad = E_PAD - E
    pad_ids = jnp.arange(pad, dtype=jnp.int32)
    src_p = jnp.concatenate([src, pad_ids % N])
    dst_p = jnp.concatenate([dst, N + pad_ids % (NPAD - N)])
    src_all = src_p.reshape(NW, NCHUNKS, CHUNK)
    dst_all = dst_p.reshape(NW, NCHUNKS, CHUNK)
    zeros_init = jnp.zeros((NPAD, H), jnp.float32)
    batch2 = batch.reshape(N, 1)

    zcat = jnp.zeros((N, 3 * H), jnp.float32)
    gs = []
    for l, (w1, b1, w2, b2, gamma, beta) in enumerate(params):
        if l == 0:
            p = _sc_agg_x(x, src_all, dst_all, zeros_init)
            zin = (x,)
        else:
            p = _sc_agg_cat[l - 1](zcat, src_all, dst_all, zeros_init)
            zin = ()
        wargs = (p, w1, b1.reshape(1, H), w2, b2.reshape(1, H),
                 gamma.reshape(1, H), beta.reshape(1, H), batch2)
        zcat, g = _tc_layers[l](zcat, *zin, *wargs)
        gs.append(g)
    return zcat, jnp.concatenate(gs, axis=1)


# speedup vs baseline: 1.0230x; 1.0199x over previous
import functools

import jax
import jax.numpy as jnp
from jax import lax
from jax.experimental import pallas as pl
from jax.experimental.pallas import tpu as pltpu
from jax.experimental.pallas import tpu_sc as plsc

N = 10000
E = 320000
H = 128
G = 64

NC = 2
NS = 16
NW = NC * NS
EPW = 10240
E_PAD = NW * EPW
CHUNK = 128
NCHUNKS = EPW // CHUNK
NSLAB = 2
HCHUNKS = NCHUNKS // NSLAB
NBUF = 2
NPAD = 10112
ZROWS = NPAD // NS
OROWS = 632

_sc_mesh = plsc.VectorSubcoreMesh(core_axis_name="c", subcore_axis_name="s")


def _make_sc_aggregate(ncols, col0):
    @functools.partial(
        pl.kernel,
        out_type=jax.ShapeDtypeStruct((NC, N, H), jnp.float32),
        mesh=_sc_mesh,
        scratch_types=[
            pltpu.VMEM((HCHUNKS, CHUNK), jnp.int32),
            pltpu.VMEM((HCHUNKS, CHUNK), jnp.int32),
            [pltpu.VMEM((CHUNK, H), jnp.float32)] * NBUF,
            [pltpu.SemaphoreType.DMA] * NBUF,
            [pltpu.SemaphoreType.DMA] * NBUF,
            pltpu.VMEM_SHARED((NPAD, H), jnp.float32),
        ],
        name=f"sc_aggregate_c{col0}",
    )
    def _sc_aggregate(z_hbm, src_hbm, dst_hbm, zeros_hbm, out_hbm,
                      sidx, didx, rows, gsems, ssems, agg):
        cid = lax.axis_index("c")
        sid = lax.axis_index("s")
        wid = cid * NS + sid

        pltpu.sync_copy(zeros_hbm.at[pl.ds(sid * ZROWS, ZROWS)],
                        agg.at[pl.ds(sid * ZROWS, ZROWS)])
        plsc.subcore_barrier()

        def zsrc(c):
            if ncols == H:
                return z_hbm.at[sidx.at[c]]
            return z_hbm.at[sidx.at[c], pl.ds(col0, H)]

        def gather(c, b):
            pltpu.async_copy(zsrc(c), rows[b], gsems[b])

        def gather_wait(c, b):
            pltpu.make_async_copy(zsrc(c), rows[b], gsems[b]).wait()

        def scatter(c, b):
            pltpu.async_copy(rows[b], agg.at[didx.at[c]], ssems[b], add=True)

        def scatter_wait(c, b):
            pltpu.make_async_copy(rows[b], agg.at[didx.at[c]],
                                  ssems[b]).wait()

        for slab in range(NSLAB):
            pltpu.sync_copy(src_hbm.at[wid, pl.ds(slab * HCHUNKS, HCHUNKS)],
                            sidx)
            pltpu.sync_copy(dst_hbm.at[wid, pl.ds(slab * HCHUNKS, HCHUNKS)],
                            didx)

            def step(c, b, first=False, last=False):
                bn = 1 - b
                if not first:
                    scatter_wait(c - 1, bn)
                if not last:
                    gather(c + 1, bn)
                gather_wait(c, b)
                scatter(c, b)

            gather(0, 0)
            step(0, 0, first=True)
            step(1, 1)

            def body(i, carry):
                step(2 * i, 0)
                step(2 * i + 1, 1)
                return carry

            lax.fori_loop(1, HCHUNKS // 2 - 1, body, 0)

            step(HCHUNKS - 2, 0)
            step(HCHUNKS - 1, 1, last=True)
            scatter_wait(HCHUNKS - 1, 1)

        plsc.subcore_barrier()

        ostart = jnp.minimum(sid * OROWS, N - OROWS)
        pltpu.sync_copy(agg.at[pl.ds(ostart, OROWS)],
                        out_hbm.at[cid, pl.ds(ostart, OROWS)])

    return _sc_aggregate


_sc_aggregate = _make_sc_aggregate(H, 0)


R = 2000
GRID = N // R


def _layer_compute(z_ref, p_ref, w1_ref, b1_ref, w2_ref, b2_ref,
                   gamma_ref, beta_ref, batch_ref,
                   zout_ref, g_ref, u_scr, stat_scr, gacc_ref):
    ph = pl.program_id(0)
    i = pl.program_id(1)

    @pl.when(ph == 0)
    def _p0():
        h = z_ref[...] + p_ref[0] + p_ref[1]
        h = jnp.maximum(
            jnp.dot(h, w1_ref[...], preferred_element_type=jnp.float32)
            + b1_ref[...], 0.0)
        h = jnp.dot(h, w2_ref[...],
                    preferred_element_type=jnp.float32) + b2_ref[...]
        u = jnp.maximum(h, 0.0)
        u_scr[pl.ds(i * R, R), :] = u

        @pl.when(i == 0)
        def _init():
            stat_scr[...] = jnp.zeros_like(stat_scr)

        stat_scr[0:1, :] += jnp.sum(u, axis=0, keepdims=True)
        stat_scr[1:2, :] += jnp.sum(u * u, axis=0, keepdims=True)

    @pl.when(ph == 1)
    def _p1():
        sums = stat_scr[...]
        mean = sums[0:1, :] * (1.0 / N)
        var = sums[1:2, :] * (1.0 / N) - mean * mean
        scale = gamma_ref[...] / jnp.sqrt(var + 1e-5)
        shift = beta_ref[...] - mean * scale
        zb = u_scr[pl.ds(i * R, R), :] * scale + shift
        zout_ref[...] = zb

        b = batch_ref[...]
        onehot = (b == lax.broadcasted_iota(jnp.int32,
                                            (R, G), 1)).astype(jnp.float32)

        @pl.when(i == 0)
        def _init():
            gacc_ref[...] = jnp.zeros_like(gacc_ref)

        gacc_ref[...] += lax.dot_general(onehot, zb, (((0,), (0,)), ((), ())),
                                         preferred_element_type=jnp.float32)

        @pl.when(i == GRID - 1)
        def _fin():
            g_ref[...] = gacc_ref[...]


_tc_layer = pl.pallas_call(
    _layer_compute,
    grid=(2, GRID),
    in_specs=[
        pl.BlockSpec((R, H), lambda p, i: (i * (1 - p), 0)),
        pl.BlockSpec((NC, R, H), lambda p, i: (0, i * (1 - p), 0)),
        pl.BlockSpec((H, H), lambda p, i: (0, 0)),
        pl.BlockSpec((1, H), lambda p, i: (0, 0)),
        pl.BlockSpec((H, H), lambda p, i: (0, 0)),
        pl.BlockSpec((1, H), lambda p, i: (0, 0)),
        pl.BlockSpec((1, H), lambda p, i: (0, 0)),
        pl.BlockSpec((1, H), lambda p, i: (0, 0)),
        pl.BlockSpec((R, 1), lambda p, i: (i * p, 0)),
    ],
    out_specs=[
        pl.BlockSpec((R, H), lambda p, i: (i * p, 0)),
        pl.BlockSpec((G, H), lambda p, i: (0, 0)),
    ],
    out_shape=[
        jax.ShapeDtypeStruct((N, H), jnp.float32),
        jax.ShapeDtypeStruct((G, H), jnp.float32),
    ],
    scratch_shapes=[
        pltpu.VMEM((N, H), jnp.float32),
        pltpu.VMEM((8, H), jnp.float32),
        pltpu.VMEM((G, H), jnp.float32),
    ],
)


def kernel(x, edge_index, batch, params):
    src = edge_index[0]
    dst = edge_index[1]
    pad = E_PAD - E
    pad_ids = jnp.arange(pad, dtype=jnp.int32)
    src_p = jnp.concatenate([src, pad_ids % N])
    dst_p = jnp.concatenate([dst, N + pad_ids % (NPAD - N)])
    src_all = src_p.reshape(NW, NCHUNKS, CHUNK)
    dst_all = dst_p.reshape(NW, NCHUNKS, CHUNK)
    zeros_init = jnp.zeros((NPAD, H), jnp.float32)
    batch2 = batch.reshape(N, 1)

    z = x
    zs = []
    gs = []
    for (w1, b1, w2, b2, gamma, beta) in params:
        p = _sc_aggregate(z, src_all, dst_all, zeros_init)
        z, g = _tc_layer(z, p, w1, b1.reshape(1, H), w2, b2.reshape(1, H),
                         gamma.reshape(1, H), beta.reshape(1, H), batch2)
        zs.append(z)
        gs.append(g)
    return jnp.concatenate(zs, axis=1), jnp.concatenate(gs, axis=1)


# speedup vs baseline: 1.0489x; 1.0253x over previous
import functools

import jax
import jax.numpy as jnp
import numpy as np
from jax import lax
from jax.experimental import pallas as pl
from jax.experimental.pallas import tpu as pltpu
from jax.experimental.pallas import tpu_sc as plsc

N = 10000
E = 320000
H = 128
G = 64

NC = 2
NS = 16
NW = NC * NS
EPW = 10240
E_PAD = NW * EPW
CHUNK = 128
NCHUNKS = EPW // CHUNK
NSLAB = 2
HCHUNKS = NCHUNKS // NSLAB
NBUF = 2
NPAD = 10112
ZROWS = NPAD // NS
OROWS = 632

_sc_mesh = plsc.VectorSubcoreMesh(core_axis_name="c", subcore_axis_name="s")


def _make_sc_aggregate(ncols, col0):
    @functools.partial(
        pl.kernel,
        out_type=jax.ShapeDtypeStruct((NC, N, H), jnp.float32),
        mesh=_sc_mesh,
        scratch_types=[
            pltpu.VMEM((HCHUNKS, CHUNK), jnp.int32),
            pltpu.VMEM((HCHUNKS, CHUNK), jnp.int32),
            [pltpu.VMEM((CHUNK, H), jnp.float32)] * NBUF,
            [pltpu.SemaphoreType.DMA] * NBUF,
            [pltpu.SemaphoreType.DMA] * NBUF,
            pltpu.SemaphoreType.DMA,
            pltpu.VMEM_SHARED((NPAD, H), jnp.float32),
        ],
        name=f"sc_aggregate_c{col0}",
    )
    def _sc_aggregate(z_hbm, src_hbm, dst_hbm, zeros_hbm, out_hbm,
                      sidx, didx, rows, gsems, ssems, zsem, agg):
        cid = lax.axis_index("c")
        sid = lax.axis_index("s")
        wid = cid * NS + sid

        zdma = pltpu.async_copy(zeros_hbm.at[pl.ds(sid * ZROWS, ZROWS)],
                                agg.at[pl.ds(sid * ZROWS, ZROWS)], zsem)

        def zsrc(c):
            if ncols == H:
                return z_hbm.at[sidx.at[c]]
            return z_hbm.at[sidx.at[c], pl.ds(col0, H)]

        def gather(c, b):
            pltpu.async_copy(zsrc(c), rows[b], gsems[b])

        def gather_wait(c, b):
            pltpu.make_async_copy(zsrc(c), rows[b], gsems[b]).wait()

        def scatter(c, b):
            pltpu.async_copy(rows[b], agg.at[didx.at[c]], ssems[b], add=True)

        def scatter_wait(c, b):
            pltpu.make_async_copy(rows[b], agg.at[didx.at[c]],
                                  ssems[b]).wait()

        for slab in range(NSLAB):
            pltpu.sync_copy(src_hbm.at[wid, pl.ds(slab * HCHUNKS, HCHUNKS)],
                            sidx)
            pltpu.sync_copy(dst_hbm.at[wid, pl.ds(slab * HCHUNKS, HCHUNKS)],
                            didx)

            def step(c, b, first=False, last=False):
                bn = 1 - b
                if not first:
                    scatter_wait(c - 1, bn)
                if not last:
                    gather(c + 1, bn)
                gather_wait(c, b)
                scatter(c, b)

            gather(0, 0)
            if slab == 0:
                zdma.wait()
                plsc.subcore_barrier()
            step(0, 0, first=True)
            step(1, 1)

            def body(i, carry):
                step(2 * i, 0)
                step(2 * i + 1, 1)
                return carry

            lax.fori_loop(1, HCHUNKS // 2 - 1, body, 0)

            step(HCHUNKS - 2, 0)
            step(HCHUNKS - 1, 1, last=True)
            scatter_wait(HCHUNKS - 1, 1)

        plsc.subcore_barrier()

        ostart = jnp.minimum(sid * OROWS, N - OROWS)
        pltpu.sync_copy(agg.at[pl.ds(ostart, OROWS)],
                        out_hbm.at[cid, pl.ds(ostart, OROWS)])

    return _sc_aggregate


_sc_aggregate = _make_sc_aggregate(H, 0)


R = 2000
GRID = N // R


def _layer_compute(z_ref, p_ref, w1_ref, b1_ref, w2_ref, b2_ref,
                   gamma_ref, beta_ref, batch_ref,
                   zout_ref, g_ref, u_scr, stat_scr, gacc_ref):
    ph = pl.program_id(0)
    i = pl.program_id(1)

    @pl.when(ph == 0)
    def _p0():
        h = z_ref[...] + p_ref[0] + p_ref[1]
        h = jnp.maximum(
            jnp.dot(h, w1_ref[...], preferred_element_type=jnp.float32)
            + b1_ref[...], 0.0)
        h = jnp.dot(h, w2_ref[...],
                    preferred_element_type=jnp.float32) + b2_ref[...]
        u = jnp.maximum(h, 0.0)
        u_scr[pl.ds(i * R, R), :] = u

        @pl.when(i == 0)
        def _init():
            stat_scr[...] = jnp.zeros_like(stat_scr)

        stat_scr[0:1, :] += jnp.sum(u, axis=0, keepdims=True)
        stat_scr[1:2, :] += jnp.sum(u * u, axis=0, keepdims=True)

    @pl.when(ph == 1)
    def _p1():
        sums = stat_scr[...]
        mean = sums[0:1, :] * (1.0 / N)
        var = sums[1:2, :] * (1.0 / N) - mean * mean
        scale = gamma_ref[...] / jnp.sqrt(var + 1e-5)
        shift = beta_ref[...] - mean * scale
        zb = u_scr[pl.ds(i * R, R), :] * scale + shift
        zout_ref[...] = zb

        b = batch_ref[...]
        onehot = (b == lax.broadcasted_iota(jnp.int32,
                                            (R, G), 1)).astype(jnp.float32)

        @pl.when(i == 0)
        def _init():
            gacc_ref[...] = jnp.zeros_like(gacc_ref)

        gacc_ref[...] += lax.dot_general(onehot, zb, (((0,), (0,)), ((), ())),
                                         preferred_element_type=jnp.float32)

        @pl.when(i == GRID - 1)
        def _fin():
            g_ref[...] = gacc_ref[...]


_tc_layer = pl.pallas_call(
    _layer_compute,
    grid=(2, GRID),
    in_specs=[
        pl.BlockSpec((R, H), lambda p, i: (i * (1 - p), 0)),
        pl.BlockSpec((NC, R, H), lambda p, i: (0, i * (1 - p), 0)),
        pl.BlockSpec((H, H), lambda p, i: (0, 0)),
        pl.BlockSpec((1, H), lambda p, i: (0, 0)),
        pl.BlockSpec((H, H), lambda p, i: (0, 0)),
        pl.BlockSpec((1, H), lambda p, i: (0, 0)),
        pl.BlockSpec((1, H), lambda p, i: (0, 0)),
        pl.BlockSpec((1, H), lambda p, i: (0, 0)),
        pl.BlockSpec((R, 1), lambda p, i: (i * p, 0)),
    ],
    out_specs=[
        pl.BlockSpec((R, H), lambda p, i: (i * p, 0)),
        pl.BlockSpec((G, H), lambda p, i: (0, 0)),
    ],
    out_shape=[
        jax.ShapeDtypeStruct((N, H), jnp.float32),
        jax.ShapeDtypeStruct((G, H), jnp.float32),
    ],
    scratch_shapes=[
        pltpu.VMEM((N, H), jnp.float32),
        pltpu.VMEM((8, H), jnp.float32),
        pltpu.VMEM((G, H), jnp.float32),
    ],
)


def kernel(x, edge_index, batch, params):
    src = edge_index[0]
    dst = edge_index[1]
    pad = E_PAD - E
    pad_ids = np.arange(pad, dtype=np.int32)
    src_p = jnp.concatenate([src, jnp.asarray(pad_ids % N)])
    dst_p = jnp.concatenate([dst, jnp.asarray(N + pad_ids % (NPAD - N))])
    src_all = src_p.reshape(NW, NCHUNKS, CHUNK)
    dst_all = dst_p.reshape(NW, NCHUNKS, CHUNK)
    zeros_init = jnp.zeros((NPAD, H), jnp.float32)
    batch2 = batch.reshape(N, 1)

    z = x
    zs = []
    gs = []
    for (w1, b1, w2, b2, gamma, beta) in params:
        p = _sc_aggregate(z, src_all, dst_all, zeros_init)
        z, g = _tc_layer(z, p, w1, b1.reshape(1, H), w2, b2.reshape(1, H),
                         gamma.reshape(1, H), beta.reshape(1, H), batch2)
        zs.append(z)
        gs.append(g)
    return jnp.concatenate(zs, axis=1), jnp.concatenate(gs, axis=1)
